# baseline (speedup 1.0000x reference)
import functools

import jax
import jax.numpy as jnp
from jax import lax
from jax.experimental import pallas as pl
from jax.experimental.pallas import tpu as pltpu
from jax.experimental.pallas import tpu_sc as plsc

V = 100000
B = 16384
D = 64
W = D + 16
ALPHA = 0.5
NS, L = 16, 16
CHUNK = B // NS
KJ = 128
NJ = CHUNK // KJ
SLAB = 3128

_mesh = plsc.VectorSubcoreMesh(core_axis_name="c", subcore_axis_name="s")

_SCRATCH = (
    [pltpu.VMEM((KJ,), jnp.int32) for _ in range(NJ)]
    + [pltpu.VMEM((KJ,), jnp.int32) for _ in range(NJ)]
    + [
        pltpu.VMEM((KJ,), jnp.int32),
        pltpu.VMEM((KJ, D), jnp.float32),
        pltpu.VMEM((KJ, D), jnp.float32),
        pltpu.VMEM((KJ, W), jnp.float32),
        pltpu.VMEM((L,), jnp.float32),
        pltpu.VMEM((NS, L), jnp.float32),
        pltpu.VMEM((D, W), jnp.float32),
        pltpu.VMEM_SHARED((NS * 6400,), jnp.int32),
        pltpu.VMEM_SHARED((B, W), jnp.float32),
        pltpu.VMEM_SHARED((NS, L), jnp.float32),
        pltpu.SemaphoreType.DMA,
        pltpu.SemaphoreType.DMA,
    ]
)


@functools.partial(
    pl.kernel,
    out_type=(
        jax.ShapeDtypeStruct((L,), jnp.float32),
        jax.ShapeDtypeStruct((V, D), jnp.float32),
    ),
    mesh=_mesh,
    compiler_params=pltpu.CompilerParams(use_tc_tiling_on_sc=False),
    scratch_types=_SCRATCH,
)
def _center_loss_sc(feat_hbm, lab_hbm, cent_hbm, loss_hbm, out_hbm, *scr):
    lab = scr[0:NJ]
    rbuf = scr[NJ:2 * NJ]
    (ibuf, crow_v, fbuf_v, wrk_v, lbuf_v, lsum_v, zrow_v,
     rep_sh, acc_sh, loss_sh, copy_sem, batch_sem) = scr[2 * NJ:]

    c = lax.axis_index("c")
    s = lax.axis_index("s")
    base = s * CHUNK

    wid = s * 2 + c
    row0 = jnp.minimum(wid * SLAB, V - SLAB)
    cdesc = pltpu.async_copy(cent_hbm.at[pl.ds(row0, SLAB)],
                             out_hbm.at[pl.ds(row0, SLAB)], copy_sem)

    zero16 = jnp.zeros((L,), jnp.float32)
    one16 = jnp.ones((L,), jnp.float32)
    iota16 = lax.iota(jnp.int32, L)

    @pl.when(c == 0)
    def _p1():
        def _zr(i, _):
            zrow_v[i // (W // L), pl.ds((i % (W // L)) * L, L)] = zero16
            return 0
        lax.fori_loop(0, D * (W // L), _zr, 0)

        for k in range(CHUNK // D):
            pltpu.sync_copy(zrow_v, acc_sh.at[pl.ds(base + k * D, D)])
        for j in range(NJ):
            pltpu.sync_copy(lab_hbm.at[pl.ds(base + j * KJ, KJ)], lab[j])
        for j in range(NJ):
            for q in range(KJ // L):
                ibuf[pl.ds(q * L, L)] = iota16 + (base + j * KJ + q * L)
            pltpu.sync_copy(ibuf, rep_sh.at[lab[j]])

    plsc.subcore_barrier()

    @pl.when(c == 0)
    def _p2():
        loss_acc = jnp.zeros((L,), jnp.float32)
        for j in range(NJ):
            pltpu.sync_copy(cent_hbm.at[lab[j]], crow_v)
            pltpu.sync_copy(feat_hbm.at[pl.ds(base + j * KJ, KJ)], fbuf_v)
            pltpu.sync_copy(rep_sh.at[lab[j]], rbuf[j])

            def _row(r, acc):
                for cc in range(D // L):
                    f = fbuf_v[r, pl.ds(cc * L, L)]
                    cr = crow_v[r, pl.ds(cc * L, L)]
                    dd = cr - f
                    acc = acc + dd * dd
                    wrk_v[r, pl.ds(cc * L, L)] = dd
                wrk_v[r, pl.ds(D, L)] = one16
                return acc
            loss_acc = lax.fori_loop(0, KJ, _row, loss_acc)

            pltpu.sync_copy(wrk_v, acc_sh.at[rbuf[j]], add=True)

        lbuf_v[pl.ds(0, L)] = loss_acc * 0.5
        pltpu.sync_copy(lbuf_v, loss_sh.at[s])

    cdesc.wait()
    plsc.subcore_barrier()

    @pl.when(c == 0)
    def _p3():
        @pl.when(s == 0)
        def _():
            pltpu.sync_copy(loss_sh, lsum_v)
            tot = jnp.zeros((L,), jnp.float32)
            for i in range(NS):
                tot = tot + lsum_v[i, pl.ds(0, L)]
            lbuf_v[pl.ds(0, L)] = tot
            pltpu.sync_copy(lbuf_v, loss_hbm)

        for j in range(NJ):
            pltpu.sync_copy(acc_sh.at[rbuf[j]], wrk_v)
            pltpu.sync_copy(cent_hbm.at[lab[j]], crow_v)

            def _row2(r, _):
                n = wrk_v[r, pl.ds(D, L)]
                scv = ALPHA / (1.0 + n)
                for cc in range(D // L):
                    cr = crow_v[r, pl.ds(cc * L, L)]
                    fbuf_v[r, pl.ds(cc * L, L)] = (
                        cr - scv * wrk_v[r, pl.ds(cc * L, L)])
                return 0
            lax.fori_loop(0, KJ, _row2, 0)

            pltpu.sync_copy(fbuf_v, out_hbm.at[lab[j]])


def kernel(features, labels, centers):
    loss_vec, out = _center_loss_sc(features, labels, centers)
    return jnp.sum(loss_vec), out
